# baseline (speedup 1.0000x reference)
import functools

import jax
import jax.numpy as jnp
from jax import lax
from jax.experimental import pallas as pl
from jax.experimental.pallas import tpu as pltpu
from jax.experimental.pallas import tpu_sc as plsc

_BATCH = 16384
_K = 64
_L = 16
_NW = 32
_BPW = _BATCH // _NW
_CHUNK = 128


def _mf_body(uidx_hbm, vidx_hbm, w_hbm, h_hbm, out_hbm,
             uidx_v, vidx_v, urows, vrows, out_v, sem):
    wid = lax.axis_index("s") * 2 + lax.axis_index("c")
    base = wid * _BPW

    pltpu.sync_copy(uidx_hbm.at[pl.ds(base, _BPW)], uidx_v)
    pltpu.sync_copy(vidx_hbm.at[pl.ds(base, _BPW)], vidx_v)

    copies = []
    for c in range(_BPW // _CHUNK):
        sl = pl.ds(c * _CHUNK, _CHUNK)
        copies.append(pltpu.async_copy(w_hbm.at[uidx_v.at[sl]], urows.at[sl], sem))
        copies.append(pltpu.async_copy(h_hbm.at[vidx_v.at[sl]], vrows.at[sl], sem))
    for cp in copies:
        cp.wait()

    lane = lax.iota(jnp.int32, _L)

    def group_body(g, carry):
        vec = jnp.zeros((_L,), jnp.float32)
        for j in range(_L):
            r = g * _L + j
            acc = urows[r, pl.ds(0, _L)] * vrows[r, pl.ds(0, _L)]
            for c in range(1, _K // _L):
                acc = acc + urows[r, pl.ds(c * _L, _L)] * vrows[r, pl.ds(c * _L, _L)]
            vec = jnp.where(lane == j, jnp.sum(acc), vec)
        out_v[pl.ds(g * _L, _L)] = 1.0 / (1.0 + jnp.exp(-vec))
        return carry

    lax.fori_loop(0, _BPW // _L, group_body, 0)

    pltpu.sync_copy(out_v, out_hbm.at[pl.ds(base, _BPW)])


_mf_kernel = functools.partial(
    pl.kernel,
    mesh=plsc.VectorSubcoreMesh(core_axis_name="c", subcore_axis_name="s"),
    out_type=jax.ShapeDtypeStruct((_BATCH,), jnp.float32),
    scratch_types=[
        pltpu.VMEM((_BPW,), jnp.int32),
        pltpu.VMEM((_BPW,), jnp.int32),
        pltpu.VMEM((_BPW, _K), jnp.float32),
        pltpu.VMEM((_BPW, _K), jnp.float32),
        pltpu.VMEM((_BPW,), jnp.float32),
        pltpu.SemaphoreType.DMA,
    ],
    compiler_params=pltpu.CompilerParams(
        needs_layout_passes=False, use_tc_tiling_on_sc=False
    ),
)(_mf_body)


def kernel(x, W, H):
    uidx = x[:, 0].astype(jnp.int32)
    vidx = x[:, 1].astype(jnp.int32)
    return _mf_kernel(uidx, vidx, W, H)
